# baseline (speedup 1.0000x reference)
import jax
import jax.numpy as jnp
from jax import lax
from jax.experimental import pallas as pl
from jax.experimental.pallas import tpu as pltpu
from jax.experimental.pallas import tpu_sc as plsc

_N = 100000
_E = 6400000

_NC = 2
_NS = 16
_L = 16

_NPAD = 100352
_NSLICE = _NPAD // _NS
_NPIECE = 8
_PIECE = _NSLICE // _NPIECE

_CHUNK = 1600
_NCHUNK = 250
_EP_TILE = _CHUNK * _NCHUNK

_NSW = 3
_NDM = 4
_GROUP = 12
_UNROLL = 10


def _body(feat_hbm, featu_hbm, edge_hbm, w_hbm, wvec_hbm,
          out_b_hbm, out_u_hbm,
          acc_sh, table_v,
          srcv0, srcv1, srcv2, dstv0, dstv1, dstv2, dstv3,
          wv0, wv1, wv2, msgv0, msgv1, msgv2, msgv3,
          wvec_v,
          in_sem0, in_sem1, in_sem2, in_sem3,
          s_sem0, s_sem1, s_sem2, s_sem3):
  srcv = (srcv0, srcv1, srcv2)
  dstv = (dstv0, dstv1, dstv2, dstv3)
  wv = (wv0, wv1, wv2)
  msgv = (msgv0, msgv1, msgv2, msgv3)
  in_sem = (in_sem0, in_sem1, in_sem2, in_sem3)
  s_sem = (s_sem0, s_sem1, s_sem2, s_sem3)
  c = lax.axis_index("c")
  s = lax.axis_index("s")
  nbase = s * _NSLICE

  @pl.when(c == 0)
  def _():
    pltpu.sync_copy(feat_hbm, table_v)

  @pl.when(c == 1)
  def _():
    pltpu.sync_copy(featu_hbm, table_v)

  def zero_body(i, _):
    sl = pl.ds(pl.multiple_of(i * _L, _L), _L)
    msgv0[sl] = jnp.zeros((_L,), jnp.float32)
    return 0

  lax.fori_loop(0, _CHUNK // _L, zero_body, 0)
  for j in range(_NPIECE):
    pltpu.sync_copy(msgv0.at[pl.ds(0, _PIECE)],
                    acc_sh.at[pl.ds(nbase + j * _PIECE, _PIECE)])

  plsc.subcore_barrier()

  ebase = s * _EP_TILE

  def start_in(j, sb, db):
    off = ebase + j * _CHUNK
    pltpu.async_copy(edge_hbm.at[pl.ds(off, _CHUNK)], srcv[sb], in_sem[db])
    pltpu.async_copy(edge_hbm.at[pl.ds(_E + off, _CHUNK)], dstv[db],
                     in_sem[db])
    pltpu.async_copy(w_hbm.at[pl.ds(off, _CHUNK)], wv[sb], in_sem[db])

  def wait_in(k, sb, db):
    off = ebase + k * _CHUNK
    pltpu.make_async_copy(edge_hbm.at[pl.ds(off, _CHUNK)], srcv[sb],
                          in_sem[db]).wait()
    pltpu.make_async_copy(edge_hbm.at[pl.ds(_E + off, _CHUNK)], dstv[db],
                          in_sem[db]).wait()
    pltpu.make_async_copy(w_hbm.at[pl.ds(off, _CHUNK)], wv[sb],
                          in_sem[db]).wait()

  def drain_scatter(db):
    pltpu.make_async_copy(msgv[db], acc_sh.at[dstv[db]], s_sem[db]).wait()

  def compute_scatter(sb, db):
    @plsc.parallel_loop(0, _CHUNK, step=_L, unroll=_UNROLL)
    def _(i):
      sl = pl.ds(pl.multiple_of(i, _L), _L)
      h = plsc.load_gather(table_v, [srcv[sb][sl]])
      msgv[db][sl] = h * wv[sb][sl]

    pltpu.async_copy(msgv[db], acc_sh.at[dstv[db]], s_sem[db], add=True)

  def step(k, b, do_drain, do_start):
    sb = b % _NSW
    db = b % _NDM
    j = b + 2
    if do_drain:
      drain_scatter(j % _NDM)
    if do_start:
      start_in(k + 2, j % _NSW, j % _NDM)
    wait_in(k, sb, db)
    compute_scatter(sb, db)

  start_in(0, 0, 0)
  start_in(1, 1, 1)

  for b in range(_GROUP):
    step(b, b, b >= 2, True)

  def group_body(p, _):
    for b in range(_GROUP):
      step(p * _GROUP + b, b, True, True)
    return 0

  lax.fori_loop(1, (_NCHUNK - 10) // _GROUP, group_body, 0)

  for b in range(10):
    k = _NCHUNK - 10 + b
    step(k, k % _GROUP, True, b < 8)
  drain_scatter((_NCHUNK - 2) % _NDM)
  drain_scatter((_NCHUNK - 1) % _NDM)

  plsc.subcore_barrier()

  pltpu.sync_copy(wvec_hbm, wvec_v)
  wb = wvec_v[0, :]
  wu = wvec_v[1, :]
  wsel = jnp.where(c == 0, wb, wu)

  for j in range(_NPIECE):
    pltpu.sync_copy(acc_sh.at[pl.ds(nbase + j * _PIECE, _PIECE)],
                    msgv0.at[pl.ds(0, _PIECE)])

    def ep_body(i, _):
      sl = pl.ds(pl.multiple_of(i * _L, _L), _L)
      msgv0[sl] = jnp.maximum(msgv0[sl] * wsel, 0.0)
      return 0

    lax.fori_loop(0, _PIECE // _L, ep_body, 0)

    @pl.when(c == 0)
    def _():
      pltpu.sync_copy(msgv0.at[pl.ds(0, _PIECE)],
                      out_b_hbm.at[pl.ds(nbase + j * _PIECE, _PIECE)])

    @pl.when(c == 1)
    def _():
      pltpu.sync_copy(msgv0.at[pl.ds(0, _PIECE)],
                      out_u_hbm.at[pl.ds(nbase + j * _PIECE, _PIECE)])


@jax.jit
def kernel(features, features_u, edge_index, edge_weight, W_belief,
           W_uncertainty):
  n = features.shape[0]

  f = features.reshape(n)
  fu = features_u.reshape(n)
  wvec = jnp.concatenate([
      jnp.broadcast_to(W_belief.reshape(1, 1), (1, _L)),
      jnp.broadcast_to(W_uncertainty.reshape(1, 1), (1, _L)),
  ], axis=0)

  mesh = plsc.VectorSubcoreMesh(core_axis_name="c", subcore_axis_name="s")
  run = pl.kernel(
      _body,
      out_type=(
          jax.ShapeDtypeStruct((_NPAD,), jnp.float32),
          jax.ShapeDtypeStruct((_NPAD,), jnp.float32),
      ),
      mesh=mesh,
      compiler_params=pltpu.CompilerParams(needs_layout_passes=False),
      scratch_types=(
          [pltpu.VMEM_SHARED((_NPAD,), jnp.float32)] +
          [pltpu.VMEM((_N,), jnp.float32)] +
          [pltpu.VMEM((_CHUNK,), jnp.int32)] * 7 +
          [pltpu.VMEM((_CHUNK,), jnp.float32)] * 7 +
          [pltpu.VMEM((2, _L), jnp.float32)] +
          [pltpu.SemaphoreType.DMA] * 8
      ),
  )
  edge_flat = edge_index.reshape(2 * _E)
  out_b, out_u = run(f, fu, edge_flat, edge_weight, wvec)
  return out_b[:n, None], out_u[:n, None]


# speedup vs baseline: 1.0148x; 1.0148x over previous
import jax
import jax.numpy as jnp
from jax import lax
from jax.experimental import pallas as pl
from jax.experimental.pallas import tpu as pltpu
from jax.experimental.pallas import tpu_sc as plsc

_N = 100000
_E = 6400000

_NC = 2
_NS = 16
_L = 16

_NPAD = 100352
_NSLICE = _NPAD // _NS
_NPIECE = 8
_PIECE = _NSLICE // _NPIECE

_CHUNK = 1600
_NCHUNK = 250
_EP_TILE = _CHUNK * _NCHUNK

_NSW = 3
_NDM = 4
_GROUP = 12
_NGRP = (_NCHUNK - 14) // _GROUP
_TAIL = _NCHUNK - 12 - _GROUP * _NGRP
_UNROLL = 5


def _body(feat_hbm, featu_hbm, edge_hbm, w_hbm, wvec_hbm,
          out_b_hbm, out_u_hbm,
          acc_sh, table_v,
          srcv0, srcv1, srcv2, dstv0, dstv1, dstv2, dstv3,
          wv0, wv1, wv2, msgv0, msgv1, msgv2, msgv3,
          wvec_v,
          in_sem0, in_sem1, in_sem2, in_sem3,
          s_sem0, s_sem1, s_sem2, s_sem3):
  srcv = (srcv0, srcv1, srcv2)
  dstv = (dstv0, dstv1, dstv2, dstv3)
  wv = (wv0, wv1, wv2)
  msgv = (msgv0, msgv1, msgv2, msgv3)
  in_sem = (in_sem0, in_sem1, in_sem2, in_sem3)
  s_sem = (s_sem0, s_sem1, s_sem2, s_sem3)
  c = lax.axis_index("c")
  s = lax.axis_index("s")
  nbase = s * _NSLICE

  @pl.when(c == 0)
  def _():
    pltpu.sync_copy(feat_hbm, table_v)

  @pl.when(c == 1)
  def _():
    pltpu.sync_copy(featu_hbm, table_v)

  def zero_body(i, _):
    sl = pl.ds(pl.multiple_of(i * _L, _L), _L)
    msgv0[sl] = jnp.zeros((_L,), jnp.float32)
    return 0

  lax.fori_loop(0, _PIECE // _L, zero_body, 0)
  for j in range(_NPIECE):
    pltpu.sync_copy(msgv0.at[pl.ds(0, _PIECE)],
                    acc_sh.at[pl.ds(nbase + j * _PIECE, _PIECE)])

  plsc.subcore_barrier()

  ebase = s * _EP_TILE

  def start_in(j, sb, db):
    off = ebase + j * _CHUNK
    pltpu.async_copy(edge_hbm.at[pl.ds(off, _CHUNK)], srcv[sb], in_sem[db])
    pltpu.async_copy(edge_hbm.at[pl.ds(_E + off, _CHUNK)], dstv[db],
                     in_sem[db])
    pltpu.async_copy(w_hbm.at[pl.ds(off, _CHUNK)], wv[sb], in_sem[db])

  def wait_in(k, sb, db):
    off = ebase + k * _CHUNK
    pltpu.make_async_copy(edge_hbm.at[pl.ds(off, _CHUNK)], srcv[sb],
                          in_sem[db]).wait()
    pltpu.make_async_copy(edge_hbm.at[pl.ds(_E + off, _CHUNK)], dstv[db],
                          in_sem[db]).wait()
    pltpu.make_async_copy(w_hbm.at[pl.ds(off, _CHUNK)], wv[sb],
                          in_sem[db]).wait()

  def drain_scatter(db):
    pltpu.make_async_copy(msgv[db], acc_sh.at[dstv[db]], s_sem[db]).wait()

  def compute_scatter(sb, db):
    @plsc.parallel_loop(0, _CHUNK, step=_L, unroll=_UNROLL)
    def _(i):
      sl = pl.ds(pl.multiple_of(i, _L), _L)
      h = plsc.load_gather(table_v, [srcv[sb][sl]])
      msgv[db][sl] = h * wv[sb][sl]

    pltpu.async_copy(msgv[db], acc_sh.at[dstv[db]], s_sem[db], add=True)

  def step(k, b, do_drain, do_start):
    sb = b % _NSW
    db = b % _NDM
    j = b + 2
    if do_drain:
      drain_scatter(j % _NDM)
    if do_start:
      start_in(k + 2, j % _NSW, j % _NDM)
    wait_in(k, sb, db)
    compute_scatter(sb, db)

  start_in(0, 0, 0)
  start_in(1, 1, 1)

  for b in range(_GROUP):
    step(b, b, b >= 2, True)

  def group_body(p, _):
    for b in range(_GROUP):
      step(p * _GROUP + b, b, True, True)
    return 0

  lax.fori_loop(1, _NGRP + 1, group_body, 0)

  for b in range(_TAIL):
    k = _NCHUNK - _TAIL + b
    step(k, k % _GROUP, True, k + 2 < _NCHUNK)
  drain_scatter((_NCHUNK - 2) % _NDM)
  drain_scatter((_NCHUNK - 1) % _NDM)

  plsc.subcore_barrier()

  pltpu.sync_copy(wvec_hbm, wvec_v)
  wb = wvec_v[0, :]
  wu = wvec_v[1, :]
  wsel = jnp.where(c == 0, wb, wu)

  for j in range(_NPIECE):
    pltpu.sync_copy(acc_sh.at[pl.ds(nbase + j * _PIECE, _PIECE)],
                    msgv0.at[pl.ds(0, _PIECE)])

    def ep_body(i, _):
      sl = pl.ds(pl.multiple_of(i * _L, _L), _L)
      msgv0[sl] = jnp.maximum(msgv0[sl] * wsel, 0.0)
      return 0

    lax.fori_loop(0, _PIECE // _L, ep_body, 0)

    @pl.when(c == 0)
    def _():
      pltpu.sync_copy(msgv0.at[pl.ds(0, _PIECE)],
                      out_b_hbm.at[pl.ds(nbase + j * _PIECE, _PIECE)])

    @pl.when(c == 1)
    def _():
      pltpu.sync_copy(msgv0.at[pl.ds(0, _PIECE)],
                      out_u_hbm.at[pl.ds(nbase + j * _PIECE, _PIECE)])


@jax.jit
def kernel(features, features_u, edge_index, edge_weight, W_belief,
           W_uncertainty):
  n = features.shape[0]

  f = features.reshape(n)
  fu = features_u.reshape(n)
  wvec = jnp.concatenate([
      jnp.broadcast_to(W_belief.reshape(1, 1), (1, _L)),
      jnp.broadcast_to(W_uncertainty.reshape(1, 1), (1, _L)),
  ], axis=0)

  mesh = plsc.VectorSubcoreMesh(core_axis_name="c", subcore_axis_name="s")
  run = pl.kernel(
      _body,
      out_type=(
          jax.ShapeDtypeStruct((_NPAD,), jnp.float32),
          jax.ShapeDtypeStruct((_NPAD,), jnp.float32),
      ),
      mesh=mesh,
      compiler_params=pltpu.CompilerParams(needs_layout_passes=False),
      scratch_types=(
          [pltpu.VMEM_SHARED((_NPAD,), jnp.float32)] +
          [pltpu.VMEM((_N,), jnp.float32)] +
          [pltpu.VMEM((_CHUNK,), jnp.int32)] * 7 +
          [pltpu.VMEM((_CHUNK,), jnp.float32)] * 7 +
          [pltpu.VMEM((2, _L), jnp.float32)] +
          [pltpu.SemaphoreType.DMA] * 8
      ),
  )
  edge_flat = edge_index.reshape(2 * _E)
  out_b, out_u = run(f, fu, edge_flat, edge_weight, wvec)
  return out_b[:n, None], out_u[:n, None]
